# baseline (speedup 1.0000x reference)
import jax
import jax.numpy as jnp
from jax.experimental import pallas as pl


def _head_kernel(ea_ref, w1_ref, b1_ref, w2_ref, b2_ref, w3_ref, b3_ref,
                 w4_ref, b4_ref, out_ref):
    ea = ea_ref[...]
    h = jnp.dot(ea, w1_ref[...], preferred_element_type=jnp.float32)
    h = jnp.maximum(h + b1_ref[...], 0.0)
    h = jnp.dot(h, w2_ref[...], preferred_element_type=jnp.float32)
    h = jnp.maximum(h + b2_ref[...], 0.0)
    h = jnp.dot(h, w3_ref[...], preferred_element_type=jnp.float32)
    h = jnp.maximum(h + b3_ref[...], 0.0)
    o = jnp.dot(h, w4_ref[...], preferred_element_type=jnp.float32)
    out_ref[...] = jax.nn.sigmoid(o + b4_ref[...])


def kernel(x, edge_index, edge_attr, params_rel, params_obj, params_w):
    E, DE = edge_attr.shape
    (W1, b1), (W2, b2), (W3, b3), (W4, b4) = params_w
    H1, H2, H3, DO = W1.shape[0], W2.shape[0], W3.shape[0], W4.shape[0]

    block = 2000
    grid = (pl.cdiv(E, block),)

    out = pl.pallas_call(
        _head_kernel,
        grid=grid,
        in_specs=[
            pl.BlockSpec((block, DE), lambda i: (i, 0)),
            pl.BlockSpec((DE, H1), lambda i: (0, 0)),
            pl.BlockSpec((1, H1), lambda i: (0, 0)),
            pl.BlockSpec((H1, H2), lambda i: (0, 0)),
            pl.BlockSpec((1, H2), lambda i: (0, 0)),
            pl.BlockSpec((H2, H3), lambda i: (0, 0)),
            pl.BlockSpec((1, H3), lambda i: (0, 0)),
            pl.BlockSpec((H3, DO), lambda i: (0, 0)),
            pl.BlockSpec((1, DO), lambda i: (0, 0)),
        ],
        out_specs=pl.BlockSpec((block, DO), lambda i: (i, 0)),
        out_shape=jax.ShapeDtypeStruct((E, DO), jnp.float32),
    )(edge_attr, W1.T, b1[None, :], W2.T, b2[None, :], W3.T, b3[None, :],
      W4.T, b4[None, :])
    return out


# speedup vs baseline: 1.3177x; 1.3177x over previous
import jax
import jax.numpy as jnp
from jax.experimental import pallas as pl


def _head_kernel(ea_ref, w1_ref, b1_ref, w2_ref, b2_ref, w3_ref, b3_ref,
                 w4_ref, b4_ref, out_ref):
    ea = ea_ref[...]
    h = jnp.dot(ea, w1_ref[...], preferred_element_type=jnp.float32)
    h = jnp.maximum(h + b1_ref[...], 0.0)
    h = jnp.dot(h, w2_ref[...], preferred_element_type=jnp.float32)
    h = jnp.maximum(h + b2_ref[...], 0.0)
    h = jnp.dot(h, w3_ref[...], preferred_element_type=jnp.float32)
    h = jnp.maximum(h + b3_ref[...], 0.0)
    o = jnp.dot(h, w4_ref[...], preferred_element_type=jnp.float32)
    out_ref[...] = jax.nn.sigmoid(o + b4_ref[...])


def kernel(x, edge_index, edge_attr, params_rel, params_obj, params_w):
    E, DE = edge_attr.shape
    (W1, b1), (W2, b2), (W3, b3), (W4, b4) = params_w
    H1, H2, H3, DO = W1.shape[0], W2.shape[0], W3.shape[0], W4.shape[0]

    block = 16000
    grid = (pl.cdiv(E, block),)

    out = pl.pallas_call(
        _head_kernel,
        grid=grid,
        in_specs=[
            pl.BlockSpec((block, DE), lambda i: (i, 0)),
            pl.BlockSpec((DE, H1), lambda i: (0, 0)),
            pl.BlockSpec((1, H1), lambda i: (0, 0)),
            pl.BlockSpec((H1, H2), lambda i: (0, 0)),
            pl.BlockSpec((1, H2), lambda i: (0, 0)),
            pl.BlockSpec((H2, H3), lambda i: (0, 0)),
            pl.BlockSpec((1, H3), lambda i: (0, 0)),
            pl.BlockSpec((H3, DO), lambda i: (0, 0)),
            pl.BlockSpec((1, DO), lambda i: (0, 0)),
        ],
        out_specs=pl.BlockSpec((block, DO), lambda i: (i, 0)),
        out_shape=jax.ShapeDtypeStruct((E, DO), jnp.float32),
    )(edge_attr, W1.T, b1[None, :], W2.T, b2[None, :], W3.T, b3[None, :],
      W4.T, b4[None, :])
    return out


# speedup vs baseline: 10.2759x; 7.7985x over previous
import jax
import jax.numpy as jnp
from jax.experimental import pallas as pl


def _head_kernel(ea_ref, w1_ref, b1_ref, w2_ref, b2_ref, w3_ref, b3_ref,
                 w4_ref, b4_ref, out_ref):
    h = jnp.dot(w1_ref[...], ea_ref[...], preferred_element_type=jnp.float32)
    h = jnp.maximum(h + b1_ref[...], 0.0)
    h = jnp.dot(w2_ref[...], h, preferred_element_type=jnp.float32)
    h = jnp.maximum(h + b2_ref[...], 0.0)
    h = jnp.dot(w3_ref[...], h, preferred_element_type=jnp.float32)
    h = jnp.maximum(h + b3_ref[...], 0.0)
    o = jnp.dot(w4_ref[...], h, preferred_element_type=jnp.float32)
    out_ref[...] = jax.nn.sigmoid(o + b4_ref[...])


def kernel(x, edge_index, edge_attr, params_rel, params_obj, params_w):
    E, DE = edge_attr.shape
    (W1, b1), (W2, b2), (W3, b3), (W4, b4) = params_w
    H1, H2, H3, DO = W1.shape[0], W2.shape[0], W3.shape[0], W4.shape[0]

    eaT = edge_attr.T

    lanes = 16000
    grid = (pl.cdiv(E, lanes),)

    out = pl.pallas_call(
        _head_kernel,
        grid=grid,
        in_specs=[
            pl.BlockSpec((DE, lanes), lambda i: (0, i)),
            pl.BlockSpec((H1, DE), lambda i: (0, 0)),
            pl.BlockSpec((H1, 1), lambda i: (0, 0)),
            pl.BlockSpec((H2, H1), lambda i: (0, 0)),
            pl.BlockSpec((H2, 1), lambda i: (0, 0)),
            pl.BlockSpec((H3, H2), lambda i: (0, 0)),
            pl.BlockSpec((H3, 1), lambda i: (0, 0)),
            pl.BlockSpec((DO, H3), lambda i: (0, 0)),
            pl.BlockSpec((DO, 1), lambda i: (0, 0)),
        ],
        out_specs=pl.BlockSpec((DO, lanes), lambda i: (0, i)),
        out_shape=jax.ShapeDtypeStruct((DO, E), jnp.float32),
    )(eaT, W1, b1[:, None], W2, b2[:, None], W3, b3[:, None],
      W4, b4[:, None])
    return out.reshape(E, DO)


# speedup vs baseline: 10.8987x; 1.0606x over previous
import jax
import jax.numpy as jnp
from jax.experimental import pallas as pl


def _head_kernel(ea_ref, w1_ref, b1_ref, w2_ref, b2_ref, w3_ref, b3_ref,
                 w4_ref, b4_ref, out_ref):
    h = jnp.dot(w1_ref[...], ea_ref[...], preferred_element_type=jnp.float32)
    h = jnp.maximum(h + b1_ref[...], 0.0)
    h = jnp.dot(w2_ref[...], h, preferred_element_type=jnp.float32)
    h = jnp.maximum(h + b2_ref[...], 0.0)
    h = jnp.dot(w3_ref[...], h, preferred_element_type=jnp.float32)
    h = jnp.maximum(h + b3_ref[...], 0.0)
    o = jnp.dot(w4_ref[...], h, preferred_element_type=jnp.float32)
    out_ref[...] = jax.nn.sigmoid(o + b4_ref[...])


def kernel(x, edge_index, edge_attr, params_rel, params_obj, params_w):
    E, DE = edge_attr.shape
    (W1, b1), (W2, b2), (W3, b3), (W4, b4) = params_w
    H1, H2, H3, DO = W1.shape[0], W2.shape[0], W3.shape[0], W4.shape[0]

    eaT = edge_attr.T

    lanes = 32000
    grid = (pl.cdiv(E, lanes),)

    out = pl.pallas_call(
        _head_kernel,
        grid=grid,
        in_specs=[
            pl.BlockSpec((DE, lanes), lambda i: (0, i)),
            pl.BlockSpec((H1, DE), lambda i: (0, 0)),
            pl.BlockSpec((H1, 1), lambda i: (0, 0)),
            pl.BlockSpec((H2, H1), lambda i: (0, 0)),
            pl.BlockSpec((H2, 1), lambda i: (0, 0)),
            pl.BlockSpec((H3, H2), lambda i: (0, 0)),
            pl.BlockSpec((H3, 1), lambda i: (0, 0)),
            pl.BlockSpec((DO, H3), lambda i: (0, 0)),
            pl.BlockSpec((DO, 1), lambda i: (0, 0)),
        ],
        out_specs=pl.BlockSpec((DO, lanes), lambda i: (0, i)),
        out_shape=jax.ShapeDtypeStruct((DO, E), jnp.float32),
    )(eaT, W1, b1[:, None], W2, b2[:, None], W3, b3[:, None],
      W4, b4[:, None])
    return out.reshape(E, DO)
